# baseline (speedup 1.0000x reference)
import functools

import jax
import jax.numpy as jnp
from jax import lax
from jax.experimental import pallas as pl
from jax.experimental.pallas import tpu as pltpu
from jax.experimental.pallas import tpu_sc as plsc

NC, NS = 2, 16
NW = NC * NS
L = 16
C = 4096


@functools.lru_cache(maxsize=None)
def _build(M, D, B):
    assert D % NW == 0 and M % L == 0
    cols_per_w = D // NW
    nchunk = B // C
    assert B % C == 0 and C % L == 0 and nchunk >= 2
    mesh = plsc.VectorSubcoreMesh(
        core_axis_name="c", subcore_axis_name="s", num_cores=NC, num_subcores=NS
    )

    @functools.partial(
        pl.kernel,
        out_type=jax.ShapeDtypeStruct((D, M), jnp.float32),
        mesh=mesh,
        compiler_params=pltpu.CompilerParams(needs_layout_passes=False),
        scratch_types=[
            pltpu.VMEM((M,), jnp.float32),
            pltpu.VMEM((B,), jnp.int32),
            pltpu.VMEM((2, C), jnp.float32),
            pltpu.SemaphoreType.DMA,
            pltpu.SemaphoreType.DMA,
            pltpu.SemaphoreType.DMA,
            pltpu.SemaphoreType.DMA,
            pltpu.SemaphoreType.DMA,
        ],
    )
    def scatter_cols(
        inp_t, idx_t, src_t, out_t, acc_v, idx_v, src2, sem_c, sem_o, sem_i,
        sem_a, sem_b,
    ):
        wid = lax.axis_index("s") * NC + lax.axis_index("c")
        src_sems = (sem_a, sem_b)

        def start_src(col, c):
            return pltpu.async_copy(
                src_t.at[col, pl.ds(c * C, C)], src2.at[c % 2], src_sems[c % 2]
            )

        col0 = 0 * NW + wid
        col_cp = pltpu.async_copy(inp_t.at[col0], acc_v, sem_c)
        idx_cp = pltpu.async_copy(idx_t.at[col0], idx_v, sem_i)
        src_cp = {0: start_src(col0, 0)}

        for k in range(cols_per_w):
            col = k * NW + wid
            col_cp.wait()
            idx_cp.wait()
            for c in range(nchunk):
                if c + 1 < nchunk:
                    src_cp[c + 1] = start_src(col, c + 1)
                elif k + 1 < cols_per_w:
                    nxt_src = start_src(col + NW, 0)
                src_cp[c].wait()

                def scat(i, carry, c=c):
                    idx16 = idx_v[pl.ds(c * C + i * L, L)]
                    val16 = src2[c % 2, pl.ds(i * L, L)]
                    plsc.addupdate_scatter(acc_v, [idx16], val16)
                    return carry

                lax.fori_loop(0, C // L, scat, 0, unroll=16)
            st_cp = pltpu.async_copy(acc_v, out_t.at[col], sem_o)
            if k + 1 < cols_per_w:
                idx_cp = pltpu.async_copy(idx_t.at[col + NW], idx_v, sem_i)
                src_cp = {0: nxt_src}
                st_cp.wait()
                col_cp = pltpu.async_copy(inp_t.at[col + NW], acc_v, sem_c)
            else:
                st_cp.wait()

    return scatter_cols


def kernel(input, index, src):
    M, D = input.shape
    B = src.shape[0]
    inp_t = input.T
    idx_t = index.astype(jnp.int32).T
    src_t = src.T
    out_t = _build(M, D, B)(inp_t, idx_t, src_t)
    return out_t.T


# speedup vs baseline: 1.0097x; 1.0097x over previous
import functools

import jax
import jax.numpy as jnp
from jax import lax
from jax.experimental import pallas as pl
from jax.experimental.pallas import tpu as pltpu
from jax.experimental.pallas import tpu_sc as plsc

NC, NS = 2, 16
NW = NC * NS
L = 16
C = 4096


@functools.lru_cache(maxsize=None)
def _build(M, D, B):
    assert D % NW == 0 and M % L == 0
    cols_per_w = D // NW
    nchunk = B // C
    assert B % C == 0 and C % L == 0 and nchunk >= 2
    mesh = plsc.VectorSubcoreMesh(
        core_axis_name="c", subcore_axis_name="s", num_cores=NC, num_subcores=NS
    )

    @functools.partial(
        pl.kernel,
        out_type=jax.ShapeDtypeStruct((D, M), jnp.float32),
        mesh=mesh,
        compiler_params=pltpu.CompilerParams(needs_layout_passes=False),
        scratch_types=[
            pltpu.VMEM((M,), jnp.float32),
            pltpu.VMEM((B,), jnp.int32),
            pltpu.VMEM((2, C), jnp.float32),
            pltpu.SemaphoreType.DMA,
            pltpu.SemaphoreType.DMA,
            pltpu.SemaphoreType.DMA,
            pltpu.SemaphoreType.DMA,
            pltpu.SemaphoreType.DMA,
        ],
    )
    def scatter_cols(
        inp_t, idx_t, src_t, out_t, acc_v, idx_v, src2, sem_c, sem_o, sem_i,
        sem_a, sem_b,
    ):
        wid = lax.axis_index("s") * NC + lax.axis_index("c")
        src_sems = (sem_a, sem_b)

        def start_src(col, c):
            return pltpu.async_copy(
                src_t.at[col, pl.ds(c * C, C)], src2.at[c % 2], src_sems[c % 2]
            )

        col0 = 0 * NW + wid
        col_cp = pltpu.async_copy(inp_t.at[col0], acc_v, sem_c)
        idx_cp = pltpu.async_copy(idx_t.at[col0], idx_v, sem_i)
        src_cp = {0: start_src(col0, 0)}

        for k in range(cols_per_w):
            col = k * NW + wid
            col_cp.wait()
            idx_cp.wait()
            for c in range(nchunk):
                if c + 1 < nchunk:
                    src_cp[c + 1] = start_src(col, c + 1)
                elif k + 1 < cols_per_w:
                    nxt_src = start_src(col + NW, 0)
                src_cp[c].wait()

                def scat(i, carry, c=c):
                    idx16 = idx_v[pl.ds(c * C + i * L, L)]
                    val16 = src2[c % 2, pl.ds(i * L, L)]
                    plsc.addupdate_scatter(acc_v, [idx16], val16)
                    return carry

                lax.fori_loop(0, C // L, scat, 0, unroll=8)
            st_cp = pltpu.async_copy(acc_v, out_t.at[col], sem_o)
            if k + 1 < cols_per_w:
                idx_cp = pltpu.async_copy(idx_t.at[col + NW], idx_v, sem_i)
                src_cp = {0: nxt_src}
                st_cp.wait()
                col_cp = pltpu.async_copy(inp_t.at[col + NW], acc_v, sem_c)
            else:
                st_cp.wait()

    return scatter_cols


def kernel(input, index, src):
    M, D = input.shape
    B = src.shape[0]
    inp_t = input.T
    idx_t = index.astype(jnp.int32).T
    src_t = src.T
    out_t = _build(M, D, B)(inp_t, idx_t, src_t)
    return out_t.T
